# baseline (speedup 1.0000x reference)
import functools

import jax
import jax.numpy as jnp
from jax import lax
from jax.experimental import pallas as pl
from jax.experimental.pallas import tpu as pltpu
from jax.experimental.pallas import tpu_sc as plsc

_N = 10000
_IN = 128
_HID = 16
_E = 320000

_ROWS = 2560
_RPT = _ROWS // 16
_MACRO = 8
_NMACRO = _RPT // _MACRO
_NPAD = 10240
_ZR = _NPAD // 16
_PR = _N // 16

_BLK = 2000

_sc_mesh = plsc.VectorSubcoreMesh(core_axis_name="c", subcore_axis_name="s")

_f32 = jnp.float32


@functools.partial(
    pl.kernel,
    out_type=(jax.ShapeDtypeStruct((2, _N, _HID), _f32),
              jax.ShapeDtypeStruct((2, _NPAD, _HID), _f32)),
    mesh=_sc_mesh,
    scratch_types=[
        pltpu.VMEM((_MACRO, 128), jnp.int32),
        pltpu.VMEM((_MACRO, 128), jnp.int32),
        pltpu.VMEM((_MACRO * 128, _HID), _f32),
        pltpu.VMEM((_ZR, _HID), _f32),
        pltpu.VMEM((_PR, _HID), _f32),
        pltpu.VMEM((_PR, _HID), _f32),
        pltpu.VMEM_SHARED((_NPAD, _HID), _f32),
        pltpu.VMEM_SHARED((_NPAD, _HID), _f32),
        pltpu.SemaphoreType.DMA,
    ],
    compiler_params=pltpu.CompilerParams(use_tc_tiling_on_sc=False),
)
def _gnn_sc(xr_hbm, xs_hbm, src_hbm, dst_hbm, h_hbm, a2_hbm,
            sidx, didx, rows, zbuf, hbuf, xsbuf, acc1, acc2, sem):
    cid = lax.axis_index("c")
    sid = lax.axis_index("s")
    xr = xr_hbm.at[cid]
    xs = xs_hbm.at[cid]
    src = src_hbm.at[cid]
    dst = dst_hbm.at[cid]
    hout = h_hbm.at[cid]
    a2out = a2_hbm.at[cid]

    zero = jnp.zeros((_HID,), _f32)

    @pl.loop(0, _ZR)
    def _(i):
        zbuf[i, :] = zero

    pltpu.sync_copy(zbuf, acc1.at[pl.ds(sid * _ZR, _ZR)])
    pltpu.sync_copy(zbuf, acc2.at[pl.ds(sid * _ZR, _ZR)])
    plsc.subcore_barrier()

    def seg(x_src_ref, acc):
        base = sid * _RPT

        @pl.loop(0, _NMACRO)
        def _(m):
            r0 = base + m * _MACRO
            pltpu.sync_copy(src.at[pl.ds(r0, _MACRO)], sidx)
            pltpu.sync_copy(dst.at[pl.ds(r0, _MACRO)], didx)
            copies = [
                pltpu.async_copy(x_src_ref.at[sidx.at[j]],
                                 rows.at[pl.ds(j * 128, 128)], sem)
                for j in range(_MACRO)
            ]
            for c in copies:
                c.wait()
            for j in range(_MACRO):
                pltpu.sync_copy(rows.at[pl.ds(j * 128, 128)],
                                acc.at[didx.at[j]], add=True)

    seg(xr, acc1)
    plsc.subcore_barrier()

    p0 = sid * _PR
    pltpu.sync_copy(acc1.at[pl.ds(p0, _PR)], hbuf)
    pltpu.sync_copy(xs.at[pl.ds(p0, _PR)], xsbuf)

    @pl.loop(0, _PR)
    def _(i):
        hbuf[i, :] = jnp.maximum(hbuf[i, :] + xsbuf[i, :], 0.0)

    pltpu.sync_copy(hbuf, hout.at[pl.ds(p0, _PR)])
    plsc.subcore_barrier()

    seg(hout, acc2)
    plsc.subcore_barrier()
    pltpu.sync_copy(acc2.at[pl.ds(sid * _ZR, _ZR)],
                    a2out.at[pl.ds(sid * _ZR, _ZR)])


def _mlp1_body(xp_ref, xl_ref, wrp, wsp, wrl, wsl, bp, bl,
               xrp_ref, xsp_ref, xrl_ref, xsl_ref):
    xp = xp_ref[...]
    xl = xl_ref[...]
    xrp_ref[...] = jnp.dot(xp, wrp[...], preferred_element_type=_f32)
    xsp_ref[...] = jnp.dot(xp, wsp[...], preferred_element_type=_f32) + bp[...]
    xrl_ref[...] = jnp.dot(xl, wrl[...], preferred_element_type=_f32)
    xsl_ref[...] = jnp.dot(xl, wsl[...], preferred_element_type=_f32) + bl[...]


def _head_body(a2p, hp, a2l, hl, wrp2, wsp2, wrl2, wsl2, b2p, b2l,
               winp, winl, bin_, wout, bout, out_ref):
    ap_t = jnp.dot(wrp2[...], winp[...], preferred_element_type=_f32)
    ap_b = jnp.dot(wsp2[...], winp[...], preferred_element_type=_f32)
    al_t = jnp.dot(wrl2[...], winl[...], preferred_element_type=_f32)
    al_b = jnp.dot(wsl2[...], winl[...], preferred_element_type=_f32)
    c = (jnp.dot(b2p[...], winp[...], preferred_element_type=_f32)
         + jnp.dot(b2l[...], winl[...], preferred_element_type=_f32)
         + bin_[...])
    a = (jnp.dot(a2p[...], ap_t, preferred_element_type=_f32)
         + jnp.dot(hp[...], ap_b, preferred_element_type=_f32)
         + jnp.dot(a2l[...], al_t, preferred_element_type=_f32)
         + jnp.dot(hl[...], al_b, preferred_element_type=_f32)
         + c)
    a = jnp.maximum(a, 0.0)
    out_ref[...] = jnp.tanh(
        jnp.dot(a, wout[...], preferred_element_type=_f32) + bout[...])


def _full(shape):
    return pl.BlockSpec(shape, lambda i: (0, 0))


def _rows(w):
    return pl.BlockSpec((_BLK, w), lambda i: (i, 0))


def _pad_edges(ei):
    ei = ei.astype(jnp.int32)
    npad = _ROWS * 128 - _E
    src = jnp.concatenate([ei[0], jnp.zeros((npad,), jnp.int32)]).reshape(_ROWS, 128)
    dst = jnp.concatenate([ei[1], jnp.full((npad,), _N, jnp.int32)]).reshape(_ROWS, 128)
    return src, dst


def kernel(protein_data, protein_edge_index, ligand_data, ligand_edge_index,
           p_Wr1, p_Ws1, p_b1, p_Wr2, p_Ws2, p_b2,
           l_Wr1, l_Ws1, l_b1, l_Wr2, l_Ws2, l_b2,
           W_in, b_in, W_out, b_out):
    sp, dp = _pad_edges(protein_edge_index)
    sl, dl = _pad_edges(ligand_edge_index)
    src_st = jnp.stack([sp, sl])
    dst_st = jnp.stack([dp, dl])

    nblk = _N // _BLK
    o16 = jax.ShapeDtypeStruct((_N, _HID), _f32)

    xrp, xsp, xrl, xsl = pl.pallas_call(
        _mlp1_body,
        grid=(nblk,),
        in_specs=[_rows(_IN), _rows(_IN),
                  _full((_IN, _HID)), _full((_IN, _HID)),
                  _full((_IN, _HID)), _full((_IN, _HID)),
                  _full((1, _HID)), _full((1, _HID))],
        out_specs=[_rows(_HID)] * 4,
        out_shape=[o16] * 4,
    )(protein_data, ligand_data, p_Wr1, p_Ws1, l_Wr1, l_Ws1,
      p_b1.reshape(1, _HID), l_b1.reshape(1, _HID))

    xr_st = jnp.stack([xrp, xrl])
    xs_st = jnp.stack([xsp, xsl])

    h_st, a2_st = _gnn_sc(xr_st, xs_st, src_st, dst_st)
    hp, hl = h_st[0], h_st[1]
    a2p, a2l = a2_st[0, :_N], a2_st[1, :_N]

    ogcn = W_in.shape[0] // 2
    ahid = W_in.shape[1]
    act = W_out.shape[1]
    out = pl.pallas_call(
        _head_body,
        grid=(nblk,),
        in_specs=[_rows(_HID)] * 4 + [
            _full((_HID, ogcn)), _full((_HID, ogcn)),
            _full((_HID, ogcn)), _full((_HID, ogcn)),
            _full((1, ogcn)), _full((1, ogcn)),
            _full((ogcn, ahid)), _full((ogcn, ahid)),
            _full((1, ahid)), _full((ahid, act)), _full((1, act))],
        out_specs=_rows(act),
        out_shape=jax.ShapeDtypeStruct((_N, act), _f32),
    )(a2p, hp, a2l, hl, p_Wr2, p_Ws2, l_Wr2, l_Ws2,
      p_b2.reshape(1, ogcn), l_b2.reshape(1, ogcn),
      W_in[:ogcn], W_in[ogcn:], b_in.reshape(1, ahid), W_out,
      b_out.reshape(1, act))
    return out


# speedup vs baseline: 1.1356x; 1.1356x over previous
import functools

import jax
import jax.numpy as jnp
from jax import lax
from jax.experimental import pallas as pl
from jax.experimental.pallas import tpu as pltpu
from jax.experimental.pallas import tpu_sc as plsc

_N = 10000
_IN = 128
_HID = 16
_E = 320000

_ROWS = 2560
_RPT = _ROWS // 16
_MACRO = 8
_NMACRO = _RPT // _MACRO
_NPAD = 10240
_ZR = _NPAD // 16
_PR = _N // 16

_BLK = 2000

_sc_mesh = plsc.VectorSubcoreMesh(core_axis_name="c", subcore_axis_name="s")

_f32 = jnp.float32


@functools.partial(
    pl.kernel,
    out_type=(jax.ShapeDtypeStruct((2, _N, _HID), _f32),
              jax.ShapeDtypeStruct((2, _NPAD, _HID), _f32)),
    mesh=_sc_mesh,
    scratch_types=[
        pltpu.VMEM((_RPT, 128), jnp.int32),
        pltpu.VMEM((_RPT, 128), jnp.int32),
        pltpu.VMEM((_MACRO * 128, _HID), _f32),
        pltpu.VMEM((_ZR, _HID), _f32),
        pltpu.VMEM((_PR, _HID), _f32),
        pltpu.VMEM((_PR, _HID), _f32),
        pltpu.VMEM_SHARED((_NPAD, _HID), _f32),
        pltpu.VMEM_SHARED((_NPAD, _HID), _f32),
        pltpu.SemaphoreType.DMA,
        pltpu.SemaphoreType.DMA,
    ],
    compiler_params=pltpu.CompilerParams(use_tc_tiling_on_sc=False),
)
def _gnn_sc(xr_hbm, xs_hbm, src_hbm, dst_hbm, h_hbm, a2_hbm,
            sidx, didx, rows, zbuf, hbuf, xsbuf, acc1, acc2, gsem, ssem):
    cid = lax.axis_index("c")
    sid = lax.axis_index("s")
    xr = xr_hbm.at[cid]
    xs = xs_hbm.at[cid]
    src = src_hbm.at[cid]
    dst = dst_hbm.at[cid]
    hout = h_hbm.at[cid]
    a2out = a2_hbm.at[cid]

    zero = jnp.zeros((_HID,), _f32)

    @pl.loop(0, _ZR)
    def _(i):
        zbuf[i, :] = zero

    pltpu.sync_copy(zbuf, acc1.at[pl.ds(sid * _ZR, _ZR)])
    pltpu.sync_copy(zbuf, acc2.at[pl.ds(sid * _ZR, _ZR)])
    plsc.subcore_barrier()

    def seg(x_src_ref, acc):
        base = sid * _RPT
        pltpu.sync_copy(src.at[pl.ds(base, _RPT)], sidx)
        pltpu.sync_copy(dst.at[pl.ds(base, _RPT)], didx)

        @pl.loop(0, _NMACRO)
        def _(m):
            i0 = m * _MACRO
            gathers = [
                pltpu.async_copy(x_src_ref.at[sidx.at[i0 + j]],
                                 rows.at[pl.ds(j * 128, 128)], gsem)
                for j in range(_MACRO)
            ]
            for c in gathers:
                c.wait()
            scatters = [
                pltpu.async_copy(rows.at[pl.ds(j * 128, 128)],
                                 acc.at[didx.at[i0 + j]], ssem, add=True)
                for j in range(_MACRO)
            ]
            for c in scatters:
                c.wait()

    seg(xr, acc1)
    plsc.subcore_barrier()

    p0 = sid * _PR
    pltpu.sync_copy(acc1.at[pl.ds(p0, _PR)], hbuf)
    pltpu.sync_copy(xs.at[pl.ds(p0, _PR)], xsbuf)

    @pl.loop(0, _PR)
    def _(i):
        hbuf[i, :] = jnp.maximum(hbuf[i, :] + xsbuf[i, :], 0.0)

    pltpu.sync_copy(hbuf, hout.at[pl.ds(p0, _PR)])
    plsc.subcore_barrier()

    seg(hout, acc2)
    plsc.subcore_barrier()
    pltpu.sync_copy(acc2.at[pl.ds(sid * _ZR, _ZR)],
                    a2out.at[pl.ds(sid * _ZR, _ZR)])


def _mlp1_body(xp_ref, xl_ref, wrp, wsp, wrl, wsl, bp, bl,
               xrp_ref, xsp_ref, xrl_ref, xsl_ref):
    xp = xp_ref[...]
    xl = xl_ref[...]
    xrp_ref[...] = jnp.dot(xp, wrp[...], preferred_element_type=_f32)
    xsp_ref[...] = jnp.dot(xp, wsp[...], preferred_element_type=_f32) + bp[...]
    xrl_ref[...] = jnp.dot(xl, wrl[...], preferred_element_type=_f32)
    xsl_ref[...] = jnp.dot(xl, wsl[...], preferred_element_type=_f32) + bl[...]


def _head_body(a2p, hp, a2l, hl, wrp2, wsp2, wrl2, wsl2, b2p, b2l,
               winp, winl, bin_, wout, bout, out_ref):
    ap_t = jnp.dot(wrp2[...], winp[...], preferred_element_type=_f32)
    ap_b = jnp.dot(wsp2[...], winp[...], preferred_element_type=_f32)
    al_t = jnp.dot(wrl2[...], winl[...], preferred_element_type=_f32)
    al_b = jnp.dot(wsl2[...], winl[...], preferred_element_type=_f32)
    c = (jnp.dot(b2p[...], winp[...], preferred_element_type=_f32)
         + jnp.dot(b2l[...], winl[...], preferred_element_type=_f32)
         + bin_[...])
    a = (jnp.dot(a2p[...], ap_t, preferred_element_type=_f32)
         + jnp.dot(hp[...], ap_b, preferred_element_type=_f32)
         + jnp.dot(a2l[...], al_t, preferred_element_type=_f32)
         + jnp.dot(hl[...], al_b, preferred_element_type=_f32)
         + c)
    a = jnp.maximum(a, 0.0)
    out_ref[...] = jnp.tanh(
        jnp.dot(a, wout[...], preferred_element_type=_f32) + bout[...])


def _full(shape):
    return pl.BlockSpec(shape, lambda i: (0, 0))


def _rows(w):
    return pl.BlockSpec((_BLK, w), lambda i: (i, 0))


def _pad_edges(ei):
    ei = ei.astype(jnp.int32)
    npad = _ROWS * 128 - _E
    src = jnp.concatenate([ei[0], jnp.zeros((npad,), jnp.int32)]).reshape(_ROWS, 128)
    dst = jnp.concatenate([ei[1], jnp.full((npad,), _N, jnp.int32)]).reshape(_ROWS, 128)
    return src, dst


def kernel(protein_data, protein_edge_index, ligand_data, ligand_edge_index,
           p_Wr1, p_Ws1, p_b1, p_Wr2, p_Ws2, p_b2,
           l_Wr1, l_Ws1, l_b1, l_Wr2, l_Ws2, l_b2,
           W_in, b_in, W_out, b_out):
    sp, dp = _pad_edges(protein_edge_index)
    sl, dl = _pad_edges(ligand_edge_index)
    src_st = jnp.stack([sp, sl])
    dst_st = jnp.stack([dp, dl])

    nblk = _N // _BLK
    o16 = jax.ShapeDtypeStruct((_N, _HID), _f32)

    xrp, xsp, xrl, xsl = pl.pallas_call(
        _mlp1_body,
        grid=(nblk,),
        in_specs=[_rows(_IN), _rows(_IN),
                  _full((_IN, _HID)), _full((_IN, _HID)),
                  _full((_IN, _HID)), _full((_IN, _HID)),
                  _full((1, _HID)), _full((1, _HID))],
        out_specs=[_rows(_HID)] * 4,
        out_shape=[o16] * 4,
    )(protein_data, ligand_data, p_Wr1, p_Ws1, l_Wr1, l_Ws1,
      p_b1.reshape(1, _HID), l_b1.reshape(1, _HID))

    xr_st = jnp.stack([xrp, xrl])
    xs_st = jnp.stack([xsp, xsl])

    h_st, a2_st = _gnn_sc(xr_st, xs_st, src_st, dst_st)
    hp, hl = h_st[0], h_st[1]
    a2p, a2l = a2_st[0, :_N], a2_st[1, :_N]

    ogcn = W_in.shape[0] // 2
    ahid = W_in.shape[1]
    act = W_out.shape[1]
    out = pl.pallas_call(
        _head_body,
        grid=(nblk,),
        in_specs=[_rows(_HID)] * 4 + [
            _full((_HID, ogcn)), _full((_HID, ogcn)),
            _full((_HID, ogcn)), _full((_HID, ogcn)),
            _full((1, ogcn)), _full((1, ogcn)),
            _full((ogcn, ahid)), _full((ogcn, ahid)),
            _full((1, ahid)), _full((ahid, act)), _full((1, act))],
        out_specs=_rows(act),
        out_shape=jax.ShapeDtypeStruct((_N, act), _f32),
    )(a2p, hp, a2l, hl, p_Wr2, p_Ws2, l_Wr2, l_Ws2,
      p_b2.reshape(1, ogcn), l_b2.reshape(1, ogcn),
      W_in[:ogcn], W_in[ogcn:], b_in.reshape(1, ahid), W_out,
      b_out.reshape(1, act))
    return out
